# baseline (speedup 1.0000x reference)
import functools

import jax
import jax.numpy as jnp
from jax import lax
from jax.experimental import pallas as pl
from jax.experimental.pallas import tpu as pltpu
from jax.experimental.pallas import tpu_sc as plsc

BATCH = 16384
EMB = 64
HID = 128

NC = 2
NS = 16
NW = NC * NS
NTAB = 1000000
SLAB = 256
NSLAB = 123
MAXBASE = 1000064 - SLAB
CHUNK = 128
SENT = BATCH
OUTROWS = BATCH + 8


def _sc_gather(user, item, ut_t, it_t):
    mesh = plsc.VectorSubcoreMesh(core_axis_name="c", subcore_axis_name="s")

    @functools.partial(
        pl.kernel,
        mesh=mesh,
        compiler_params=pltpu.CompilerParams(needs_layout_passes=False),
        out_type=[
            jax.ShapeDtypeStruct((OUTROWS, 128), jnp.float32),
            jax.ShapeDtypeStruct((OUTROWS, 128), jnp.float32),
        ],
        scratch_types=[
            pltpu.VMEM((BATCH,), jnp.int32),
            pltpu.VMEM((BATCH,), jnp.int32),
            pltpu.VMEM((BATCH,), jnp.int32),
            pltpu.VMEM((EMB, SLAB), jnp.float32),
            pltpu.VMEM((EMB, SLAB), jnp.float32),
            pltpu.VMEM((CHUNK, 128), jnp.float32),
            pltpu.VMEM((1, 128), jnp.int32),
            pltpu.VMEM((16,), jnp.int32),
            pltpu.VMEM((16,), jnp.int32),
            pltpu.SemaphoreType.DMA,
            pltpu.SemaphoreType.DMA,
            pltpu.SemaphoreType.DMA,
        ],
    )
    def gather_kernel(user_hbm, item_hbm, ut_hbm, it_hbm, uo_hbm, io_hbm,
                      idxall, match_r, match_j, bank_a, bank_b,
                      staging, jbuf, win_r, win_j, sem_a, sem_b, sem_o):
        wid = lax.axis_index("s") * NC + lax.axis_index("c")
        iota16 = lax.iota(jnp.int32, 16)
        start_w = wid * (NSLAB * SLAB)
        end_w = jnp.minimum(start_w + NSLAB * SLAB, NTAB)

        def slab_base(t):
            return pl.multiple_of(
                jnp.minimum(start_w + t * SLAB, MAXBASE), 128)

        def reset_jbuf():
            for mg in range(8):
                jbuf[0, pl.ds(16 * mg, 16)] = jnp.broadcast_to(SENT, (16,))

        def do_table(idx_hbm, tbl_hbm, out_hbm):
            pltpu.sync_copy(idx_hbm, idxall)
            reset_jbuf()

            def scan_body(i, cntm):
                v = idxall[pl.ds(i * 16, 16)]
                m = (v >= start_w) & (v < end_w)
                mi = m.astype(jnp.int32)
                pos = cntm + plsc.cumsum(mi) - mi
                plsc.store_scatter(match_r, [pos], v, mask=m)
                plsc.store_scatter(match_j, [pos], iota16 + 16 * i, mask=m)
                return cntm + plsc.all_reduce_population_count(m)[0]

            cntm = lax.fori_loop(0, BATCH // 16, scan_body, 0)
            nmv = (cntm + 15) >> 4

            def flush():
                pltpu.async_copy(staging, out_hbm.at[jbuf.at[0]], sem_o).wait()
                reset_jbuf()

            def process(t, bank, cnt):
                base = slab_base(t)

                def win_body(v, cnt):
                    rvec = match_r[pl.ds(v * 16, 16)]
                    valid = (iota16 + 16 * v) < cntm
                    m = valid & (rvec >= base) & (rvec < base + SLAB)
                    mi = m.astype(jnp.int32)
                    wpos = plsc.cumsum(mi) - mi
                    plsc.store_scatter(win_r, [wpos], rvec, mask=m)
                    jm = match_j[pl.ds(v * 16, 16)]
                    plsc.store_scatter(win_j, [wpos], jm, mask=m)
                    hc = plsc.all_reduce_population_count(m)[0]

                    def hit_body(h, cnt):
                        hsp = jnp.broadcast_to(h, (16,))
                        colsp = plsc.load_gather(win_r, [hsp]) - base
                        jsp = plsc.load_gather(win_j, [hsp])
                        pos = cnt & (CHUNK - 1)
                        for mg in range(EMB // 16):
                            vm = plsc.load_gather(
                                bank, [iota16 + 16 * mg, colsp])
                            staging[pos, pl.ds(16 * mg, 16)] = vm
                        plsc.store_scatter(
                            jbuf.at[0], [jnp.broadcast_to(pos, (16,))],
                            jsp, mask=iota16 == 0)
                        cnt = cnt + 1

                        @pl.when((cnt & (CHUNK - 1)) == 0)
                        def _():
                            flush()

                        return cnt

                    return lax.fori_loop(0, hc, hit_body, cnt)

                return lax.fori_loop(0, nmv, win_body, cnt)

            pltpu.async_copy(
                tbl_hbm.at[:, pl.ds(slab_base(0), SLAB)], bank_a, sem_a)
            pltpu.async_copy(
                tbl_hbm.at[:, pl.ds(slab_base(1), SLAB)], bank_b, sem_b)

            def slab_pair(u, cnt):
                t0 = 2 * u
                pltpu.make_async_copy(
                    tbl_hbm.at[:, pl.ds(0, SLAB)], bank_a, sem_a).wait()
                cnt = process(t0, bank_a, cnt)
                pltpu.async_copy(
                    tbl_hbm.at[:, pl.ds(slab_base(jnp.minimum(t0 + 2, NSLAB)),
                                        SLAB)], bank_a, sem_a)
                pltpu.make_async_copy(
                    tbl_hbm.at[:, pl.ds(0, SLAB)], bank_b, sem_b).wait()
                cnt = process(t0 + 1, bank_b, cnt)
                pltpu.async_copy(
                    tbl_hbm.at[:, pl.ds(slab_base(jnp.minimum(t0 + 3, NSLAB)),
                                        SLAB)], bank_b, sem_b)
                return cnt

            cnt = lax.fori_loop(0, (NSLAB + 1) // 2, slab_pair, 0)
            pltpu.make_async_copy(
                tbl_hbm.at[:, pl.ds(0, SLAB)], bank_a, sem_a).wait()
            pltpu.make_async_copy(
                tbl_hbm.at[:, pl.ds(0, SLAB)], bank_b, sem_b).wait()

            @pl.when((cnt & (CHUNK - 1)) != 0)
            def _():
                flush()

        do_table(user_hbm, ut_hbm, uo_hbm)
        do_table(item_hbm, it_hbm, io_hbm)

    return gather_kernel(user, item, ut_t, it_t)


BLK = 2048


def _mlp_body(u_ref, i_ref, w1u_ref, w1i_ref, b1_ref, w2_ref, b2_ref, o_ref):
    xu = lax.dot_general(u_ref[:, :EMB], w1u_ref[...], (((1,), (0,)), ((), ())),
                         preferred_element_type=jnp.float32)
    xi = lax.dot_general(i_ref[:, :EMB], w1i_ref[...], (((1,), (0,)), ((), ())),
                         preferred_element_type=jnp.float32)
    h = jnp.maximum(xu + xi + b1_ref[...], 0.0)
    y = lax.dot_general(h, w2_ref[...], (((1,), (0,)), ((), ())),
                        preferred_element_type=jnp.float32)
    o_ref[...] = 4.0 * jax.nn.sigmoid(y[:, 0:1] + b2_ref[0]) + 1.0


def _tc_mlp(uemb, iemb, w1u, w1i, b1, w2, b2):
    grid = (BATCH // BLK,)
    return pl.pallas_call(
        _mlp_body,
        grid=grid,
        in_specs=[
            pl.BlockSpec((BLK, 128), lambda b: (b, 0)),
            pl.BlockSpec((BLK, 128), lambda b: (b, 0)),
            pl.BlockSpec((EMB, HID), lambda b: (0, 0)),
            pl.BlockSpec((EMB, HID), lambda b: (0, 0)),
            pl.BlockSpec((1, HID), lambda b: (0, 0)),
            pl.BlockSpec((HID, 128), lambda b: (0, 0)),
            pl.BlockSpec(memory_space=pltpu.SMEM),
        ],
        out_specs=pl.BlockSpec((BLK, 1), lambda b: (b, 0)),
        out_shape=jax.ShapeDtypeStruct((BATCH, 1), jnp.float32),
    )(uemb, iemb, w1u, w1i, b1, w2, b2)


@jax.jit
def _run(user, item, user_table, item_table, W1, b1, W2, b2):
    uemb, iemb = _sc_gather(user.astype(jnp.int32), item.astype(jnp.int32),
                            user_table.T, item_table.T)
    w1u = W1[:, :EMB].T
    w1i = W1[:, EMB:].T
    w2pad = jnp.zeros((HID, 128), jnp.float32).at[:, 0].set(W2[0])
    out = _tc_mlp(uemb, iemb, w1u, w1i, b1.reshape(1, HID), w2pad, b2)
    return out.reshape(-1)


def kernel(user, item, user_table, item_table, W1, b1, W2, b2):
    return _run(user, item, user_table, item_table, W1, b1, W2, b2)
